# baseline (speedup 1.0000x reference)
import functools

import jax
import jax.numpy as jnp
from jax import lax
from jax.experimental import pallas as pl
from jax.experimental.pallas import tpu as pltpu
from jax.experimental.pallas import tpu_sc as plsc

N = 10000
E = 160000
D = 128
G = 256
EI = 8192
NP = 10240
EP = 163840

BN = 1000
BE = 1280
NEB = EP // BE
NEB_REAL = E // BE


def _node_proj_body(x_ref, wn_ref, wma_ref, atta_ref, h_ref, ha_ref, hs_ref):
    h = jnp.maximum(jnp.dot(x_ref[...], wn_ref[...],
                            preferred_element_type=jnp.float32), 0.0)
    h_ref[...] = h
    ha_ref[...] = jnp.dot(h, wma_ref[...], preferred_element_type=jnp.float32)
    hs_ref[...] = jnp.dot(h, atta_ref[...], preferred_element_type=jnp.float32)


def _node_proj(x, Wn, WmA, attA):
    return pl.pallas_call(
        _node_proj_body,
        grid=(N // BN,),
        in_specs=[
            pl.BlockSpec((BN, D), lambda i: (i, 0)),
            pl.BlockSpec((D, D), lambda i: (0, 0)),
            pl.BlockSpec((D, D), lambda i: (0, 0)),
            pl.BlockSpec((D, 1), lambda i: (0, 0)),
        ],
        out_specs=[
            pl.BlockSpec((BN, D), lambda i: (i, 0)),
            pl.BlockSpec((BN, D), lambda i: (i, 0)),
            pl.BlockSpec((BN, 1), lambda i: (i, 0)),
        ],
        out_shape=[
            jax.ShapeDtypeStruct((N, D), jnp.float32),
            jax.ShapeDtypeStruct((N, D), jnp.float32),
            jax.ShapeDtypeStruct((N, 1), jnp.float32),
        ],
    )(x, Wn, WmA, attA)


def _edge_proj_body(ea_ref, we_ref, wmb_ref, out_ref):
    i = pl.program_id(0)
    e = jnp.maximum(jnp.dot(ea_ref[...], we_ref[...],
                            preferred_element_type=jnp.float32), 0.0)
    eb = jnp.dot(e, wmb_ref[...], preferred_element_type=jnp.float32)
    out_ref[...] = jnp.where(i < NEB_REAL, eb, jnp.zeros_like(eb))


def _edge_proj(edge_attr, We, WmB):
    de = edge_attr.shape[1]
    return pl.pallas_call(
        _edge_proj_body,
        grid=(NEB,),
        in_specs=[
            pl.BlockSpec((BE, de), lambda i: (jnp.minimum(i, NEB_REAL - 1), 0)),
            pl.BlockSpec((de, D), lambda i: (0, 0)),
            pl.BlockSpec((D, D), lambda i: (0, 0)),
        ],
        out_specs=pl.BlockSpec((BE, D), lambda i: (i, 0)),
        out_shape=jax.ShapeDtypeStruct((EP, D), jnp.float32),
    )(edge_attr, We, WmB)


def _final_body(h_ref, acc_ref, den_ref, wu_ref, batch_ref, g_ref):
    acc = acc_ref[0] + acc_ref[1]
    den = jnp.sum(den_ref[...], axis=1)
    agg = acc / (den + 1e-30)[:, None]
    h2 = jnp.maximum(
        h_ref[...] + jnp.dot(agg, wu_ref[...],
                             preferred_element_type=jnp.float32), 0.0)
    b = batch_ref[0, 0]
    onehot = (b[:, None] == lax.broadcasted_iota(jnp.int32, (1, G), 1)
              ).astype(jnp.float32)
    gpart = lax.dot_general(onehot, h2, (((0,), (0,)), ((), ())),
                            preferred_element_type=jnp.float32)

    @pl.when(pl.program_id(0) == 0)
    def _():
        g_ref[...] = jnp.zeros_like(g_ref)

    g_ref[...] += gpart


def _final_readout(h, acc2, den32, Wu, batch3):
    return pl.pallas_call(
        _final_body,
        grid=(N // BN,),
        in_specs=[
            pl.BlockSpec((BN, D), lambda i: (i, 0)),
            pl.BlockSpec((2, BN, D), lambda i: (0, i, 0)),
            pl.BlockSpec((BN, 32), lambda i: (i, 0)),
            pl.BlockSpec((D, D), lambda i: (0, 0)),
            pl.BlockSpec((1, 1, BN), lambda i: (i, 0, 0)),
        ],
        out_specs=pl.BlockSpec((G, D), lambda i: (0, 0)),
        out_shape=jax.ShapeDtypeStruct((G, D), jnp.float32),
    )(h, acc2, den32, Wu, batch3)


BEI = 1024


def _rgcn_body(gs_ref, gt_ref, wsd_ref, wst_ref, wdt_ref, wtd_ref, e_ref,
               out_ref):
    r, c = pl.program_id(0), pl.program_id(1)
    is_d = (r % 2) == 0
    hsrc = jnp.where(is_d, gt_ref[...], gs_ref[...])
    W = jnp.where(is_d, wtd_ref[...], wdt_ref[...])
    proj = jnp.dot(hsrc, W, preferred_element_type=jnp.float32)
    e0 = e_ref[0, 0]
    e1 = e_ref[0, 1]
    iota = lax.broadcasted_iota(jnp.int32, (1, G), 1)
    oh0 = (e0[:, None] == iota).astype(jnp.float32)
    oh1 = (e1[:, None] == iota).astype(jnp.float32)
    gathered = jnp.dot(oh0, proj, preferred_element_type=jnp.float32)
    msg = lax.dot_general(oh1, gathered, (((0,), (0,)), ((), ())),
                          preferred_element_type=jnp.float32)

    @pl.when(c == 0)
    def _():
        out_ref[...] = jnp.zeros_like(out_ref)

    out_ref[...] += msg[None]

    @pl.when(c == (EI // BEI) - 1)
    def _():
        hb = jnp.where(is_d, gs_ref[...], gt_ref[...])
        Wb = jnp.where(is_d, wsd_ref[...], wst_ref[...])
        base = jnp.dot(hb, Wb, preferred_element_type=jnp.float32)
        out_ref[...] = jnp.maximum(base[None] + out_ref[...], 0.0)


def _rgcn(gs, gt, Wsd, Wst, Wdt, Wtd, eall):
    return pl.pallas_call(
        _rgcn_body,
        grid=(4, EI // BEI),
        in_specs=[
            pl.BlockSpec((G, D), lambda r, c: (0, 0)),
            pl.BlockSpec((G, D), lambda r, c: (0, 0)),
            pl.BlockSpec((D, D), lambda r, c: (0, 0)),
            pl.BlockSpec((D, D), lambda r, c: (0, 0)),
            pl.BlockSpec((D, D), lambda r, c: (0, 0)),
            pl.BlockSpec((D, D), lambda r, c: (0, 0)),
            pl.BlockSpec((1, 2, BEI), lambda r, c: (r, 0, c)),
        ],
        out_specs=pl.BlockSpec((1, G, D), lambda r, c: (r, 0, 0)),
        out_shape=jax.ShapeDtypeStruct((4, G, D), jnp.float32),
    )(gs, gt, Wsd, Wst, Wdt, Wtd, eall)


NC, NS = 2, 16
NW = NC * NS
CH = 32
NBUF = 2
TPW = EP // NW
NCH = TPW // CH
RPT = NP // NS

_sc_mesh = plsc.VectorSubcoreMesh(core_axis_name="c", subcore_axis_name="s")


@functools.partial(
    pl.kernel,
    out_type=[jax.ShapeDtypeStruct((2, NP, D), jnp.float32),
              jax.ShapeDtypeStruct((NW, NP), jnp.float32)],
    mesh=_sc_mesh,
    compiler_params=pltpu.CompilerParams(needs_layout_passes=False),
    scratch_types=[
        pltpu.VMEM((NBUF, CH), jnp.float32),
        pltpu.VMEM((NP,), jnp.float32),
        pltpu.VMEM((D,), jnp.float32),
        pltpu.VMEM((NBUF, CH), jnp.int32),
        pltpu.VMEM((NBUF, CH), jnp.int32),
        pltpu.VMEM((NBUF, CH, D), jnp.float32),
        pltpu.VMEM((NBUF, CH, D), jnp.float32),
        pltpu.VMEM((NBUF, CH, D), jnp.float32),
        pltpu.VMEM((NBUF, CH), jnp.int32),
        pltpu.VMEM_SHARED((NP, D), jnp.float32),
        pltpu.SemaphoreType.DMA,
        pltpu.SemaphoreType.DMA,
        pltpu.SemaphoreType.DMA,
        pltpu.SemaphoreType.DMA,
    ],
)
def _sc_edge(hA, ha, eB, srcp, dstp, attB, acc_out, den_out,
             hav2, den_tab, attb_v, src2, dst2, hag2, ebv2, wmv2, sdst,
             accum_sh, sem0, sem1, sem2, sem3):
    c = lax.axis_index("c")
    s = lax.axis_index("s")
    wid = s * NC + c
    sems = [sem0, sem1]
    ssems = [sem2, sem3]

    pltpu.sync_copy(attB, attb_v)

    def _zrow(r, carry):
        for f in range(8):
            wmv2[0, r, pl.ds(f * 16, 16)] = jnp.zeros((16,), jnp.float32)
        return carry
    lax.fori_loop(0, CH, _zrow, 0)

    def _zden(i, carry):
        den_tab[pl.ds(i * 16, 16)] = jnp.zeros((16,), jnp.float32)
        return carry
    lax.fori_loop(0, NP // 16, _zden, 0)

    for j in range(RPT // CH):
        pltpu.sync_copy(wmv2.at[0], accum_sh.at[pl.ds(s * RPT + j * CH, CH)])
    plsc.subcore_barrier()

    attb_regs = [attb_v[pl.ds(f * 16, 16)] for f in range(8)]
    iota16 = lax.iota(jnp.int32, 16)

    def _start(ci, b):
        base = wid * TPW + ci * CH
        pltpu.sync_copy(srcp.at[pl.ds(base, CH)], src2.at[b])
        pltpu.sync_copy(dstp.at[pl.ds(base, CH)], dst2.at[b])
        pltpu.async_copy(hA.at[src2.at[b]], hag2.at[b], sems[b])
        pltpu.async_copy(eB.at[pl.ds(base, CH)], ebv2.at[b], sems[b])
        pltpu.async_copy(ha.at[dst2.at[b]], hav2.at[b], sems[b])

    def _wait(ci, b):
        base = wid * TPW + ci * CH
        pltpu.make_async_copy(hA.at[src2.at[b]], hag2.at[b], sems[b]).wait()
        pltpu.make_async_copy(eB.at[pl.ds(base, CH)], ebv2.at[b],
                              sems[b]).wait()
        pltpu.make_async_copy(ha.at[dst2.at[b]], hav2.at[b], sems[b]).wait()

    def _compute(b):
        hag, ebv, dst_v = hag2.at[b], ebv2.at[b], dst2.at[b]
        wmv, sdst_b, hav = wmv2.at[b], sdst.at[b], hav2.at[b]
        for g in range(CH // 16):
            dstv = dst_v[pl.ds(g * 16, 16)]
            hadv = hav[pl.ds(g * 16, 16)]
            exacc = jnp.zeros((16,), jnp.float32)
            for j in range(16):
                e = g * 16 + j
                acc = None
                for f in range(8):
                    v = jnp.maximum(
                        hag[e, pl.ds(f * 16, 16)] + ebv[e, pl.ds(f * 16, 16)],
                        0.0)
                    t = v * attb_regs[f]
                    acc = t if acc is None else acc + t
                sdot = jnp.sum(acc)
                l = hadv[j] + sdot
                l = jnp.where(l >= 0, l, 0.2 * l)
                exv = jnp.exp(jnp.full((16,), l, jnp.float32))
                exacc = jnp.where(iota16 == j, exv, exacc)
                for f in range(8):
                    v = jnp.maximum(
                        hag[e, pl.ds(f * 16, 16)] + ebv[e, pl.ds(f * 16, 16)],
                        0.0)
                    wmv[e, pl.ds(f * 16, 16)] = v * exv
            plsc.addupdate_scatter(den_tab, [dstv], exacc)
            sdst_b[pl.ds(g * 16, 16)] = dstv
        pltpu.async_copy(wmv, accum_sh.at[sdst_b], ssems[b], add=True)

    def _scatter_wait(b):
        pltpu.make_async_copy(wmv2.at[b], accum_sh.at[sdst.at[b]],
                              ssems[b]).wait()

    for b in range(NBUF):
        _start(b, b)

    def _body(i, carry):
        for b in range(NBUF):
            cc = i * NBUF + b
            _wait(cc, b)

            @pl.when(cc >= NBUF)
            def _():
                _scatter_wait(b)
            _compute(b)

            @pl.when(cc + NBUF < NCH)
            def _():
                _start(cc + NBUF, b)
        return carry
    lax.fori_loop(0, NCH // NBUF, _body, 0)
    for b in range(NBUF):
        _scatter_wait(b)

    pltpu.sync_copy(den_tab, den_out.at[wid])
    plsc.subcore_barrier()
    for j in range(RPT // CH):
        r0 = s * RPT + j * CH
        pltpu.sync_copy(accum_sh.at[pl.ds(r0, CH)],
                        acc_out.at[c, pl.ds(r0, CH)])


def _attentive_fp(x, edge_index, edge_attr, batch, Wn, We, Wm, att, Wu):
    WmA, WmB = Wm[:D], Wm[D:]
    attA, attB = att[:D], att[D:, 0]
    h, hA, ha = _node_proj(x, Wn, WmA, attA)
    eB = _edge_proj(edge_attr, We, WmB)
    src = edge_index[0].astype(jnp.int32)
    dst = edge_index[1].astype(jnp.int32)
    srcp = jnp.concatenate([src, jnp.zeros((EP - E,), jnp.int32)])
    dstp = jnp.concatenate([dst, jnp.full((EP - E,), N, jnp.int32)])
    hap = jnp.concatenate([ha[:, 0], jnp.zeros((NP - N,), jnp.float32)])
    acc2, den32 = _sc_edge(hA, hap, eB, srcp, dstp, attB)
    batch3 = batch.astype(jnp.int32).reshape(N // BN, 1, BN)
    return _final_readout(h, acc2, den32.T, Wu, batch3)


def kernel(x_small, edge_index_small, edge_attr_small, batch_small,
           x_target, edge_index_target, edge_attr_target, batch_target,
           pos_edge_dt, pos_edge_td, neg_edge_dt, neg_edge_td,
           Wn_s, We_s, Wm_s, att_s, Wu_s,
           Wn_t, We_t, Wm_t, att_t, Wu_t,
           Wsd, Wst, Wdt, Wtd):
    gs = _attentive_fp(x_small, edge_index_small, edge_attr_small,
                       batch_small, Wn_s, We_s, Wm_s, att_s, Wu_s)
    gt = _attentive_fp(x_target, edge_index_target, edge_attr_target,
                       batch_target, Wn_t, We_t, Wm_t, att_t, Wu_t)
    eall = jnp.stack([pos_edge_td, pos_edge_dt, neg_edge_td, neg_edge_dt]
                     ).astype(jnp.int32)
    return _rgcn(gs, gt, Wsd, Wst, Wdt, Wtd, eall)


# speedup vs baseline: 1.2641x; 1.2641x over previous
import functools

import jax
import jax.numpy as jnp
from jax import lax
from jax.experimental import pallas as pl
from jax.experimental.pallas import tpu as pltpu
from jax.experimental.pallas import tpu_sc as plsc

N = 10000
E = 160000
D = 128
G = 256
EI = 8192
NP = 10240
EP = 163840

BN = 1000
BE = 1280
NEB = EP // BE
NEB_REAL = E // BE


def _node_proj_body(x_ref, wn_ref, wma_ref, atta_ref, h_ref, ha_ref, hs_ref):
    h = jnp.maximum(jnp.dot(x_ref[...], wn_ref[...],
                            preferred_element_type=jnp.float32), 0.0)
    h_ref[...] = h
    ha_ref[...] = jnp.dot(h, wma_ref[...], preferred_element_type=jnp.float32)
    hs_ref[...] = jnp.dot(h, atta_ref[...], preferred_element_type=jnp.float32)


def _node_proj(x, Wn, WmA, attA):
    return pl.pallas_call(
        _node_proj_body,
        grid=(N // BN,),
        in_specs=[
            pl.BlockSpec((BN, D), lambda i: (i, 0)),
            pl.BlockSpec((D, D), lambda i: (0, 0)),
            pl.BlockSpec((D, D), lambda i: (0, 0)),
            pl.BlockSpec((D, 1), lambda i: (0, 0)),
        ],
        out_specs=[
            pl.BlockSpec((BN, D), lambda i: (i, 0)),
            pl.BlockSpec((BN, D), lambda i: (i, 0)),
            pl.BlockSpec((BN, 1), lambda i: (i, 0)),
        ],
        out_shape=[
            jax.ShapeDtypeStruct((N, D), jnp.float32),
            jax.ShapeDtypeStruct((N, D), jnp.float32),
            jax.ShapeDtypeStruct((N, 1), jnp.float32),
        ],
    )(x, Wn, WmA, attA)


def _edge_proj_body(ea_ref, we_ref, wmb_ref, out_ref):
    i = pl.program_id(0)
    e = jnp.maximum(jnp.dot(ea_ref[...], we_ref[...],
                            preferred_element_type=jnp.float32), 0.0)
    eb = jnp.dot(e, wmb_ref[...], preferred_element_type=jnp.float32)
    out_ref[...] = jnp.where(i < NEB_REAL, eb, jnp.zeros_like(eb))


def _edge_proj(edge_attr, We, WmB):
    de = edge_attr.shape[1]
    return pl.pallas_call(
        _edge_proj_body,
        grid=(NEB,),
        in_specs=[
            pl.BlockSpec((BE, de), lambda i: (jnp.minimum(i, NEB_REAL - 1), 0)),
            pl.BlockSpec((de, D), lambda i: (0, 0)),
            pl.BlockSpec((D, D), lambda i: (0, 0)),
        ],
        out_specs=pl.BlockSpec((BE, D), lambda i: (i, 0)),
        out_shape=jax.ShapeDtypeStruct((EP, D), jnp.float32),
    )(edge_attr, We, WmB)


def _final_body(h_ref, acc_ref, den_ref, wu_ref, batch_ref, g_ref):
    acc = acc_ref[0] + acc_ref[1]
    den = jnp.sum(den_ref[...], axis=1)
    agg = acc / (den + 1e-30)[:, None]
    h2 = jnp.maximum(
        h_ref[...] + jnp.dot(agg, wu_ref[...],
                             preferred_element_type=jnp.float32), 0.0)
    b = batch_ref[0, 0]
    onehot = (b[:, None] == lax.broadcasted_iota(jnp.int32, (1, G), 1)
              ).astype(jnp.float32)
    gpart = lax.dot_general(onehot, h2, (((0,), (0,)), ((), ())),
                            preferred_element_type=jnp.float32)

    @pl.when(pl.program_id(0) == 0)
    def _():
        g_ref[...] = jnp.zeros_like(g_ref)

    g_ref[...] += gpart


def _final_readout(h, acc2, den32, Wu, batch3):
    return pl.pallas_call(
        _final_body,
        grid=(N // BN,),
        in_specs=[
            pl.BlockSpec((BN, D), lambda i: (i, 0)),
            pl.BlockSpec((2, BN, D), lambda i: (0, i, 0)),
            pl.BlockSpec((BN, 32), lambda i: (i, 0)),
            pl.BlockSpec((D, D), lambda i: (0, 0)),
            pl.BlockSpec((1, 1, BN), lambda i: (i, 0, 0)),
        ],
        out_specs=pl.BlockSpec((G, D), lambda i: (0, 0)),
        out_shape=jax.ShapeDtypeStruct((G, D), jnp.float32),
    )(h, acc2, den32, Wu, batch3)


BEI = 1024


def _rgcn_body(gs_ref, gt_ref, wsd_ref, wst_ref, wdt_ref, wtd_ref, e_ref,
               out_ref):
    r, c = pl.program_id(0), pl.program_id(1)
    is_d = (r % 2) == 0
    hsrc = jnp.where(is_d, gt_ref[...], gs_ref[...])
    W = jnp.where(is_d, wtd_ref[...], wdt_ref[...])
    proj = jnp.dot(hsrc, W, preferred_element_type=jnp.float32)
    e0 = e_ref[0, 0]
    e1 = e_ref[0, 1]
    iota = lax.broadcasted_iota(jnp.int32, (1, G), 1)
    oh0 = (e0[:, None] == iota).astype(jnp.float32)
    oh1 = (e1[:, None] == iota).astype(jnp.float32)
    gathered = jnp.dot(oh0, proj, preferred_element_type=jnp.float32)
    msg = lax.dot_general(oh1, gathered, (((0,), (0,)), ((), ())),
                          preferred_element_type=jnp.float32)

    @pl.when(c == 0)
    def _():
        out_ref[...] = jnp.zeros_like(out_ref)

    out_ref[...] += msg[None]

    @pl.when(c == (EI // BEI) - 1)
    def _():
        hb = jnp.where(is_d, gs_ref[...], gt_ref[...])
        Wb = jnp.where(is_d, wsd_ref[...], wst_ref[...])
        base = jnp.dot(hb, Wb, preferred_element_type=jnp.float32)
        out_ref[...] = jnp.maximum(base[None] + out_ref[...], 0.0)


def _rgcn(gs, gt, Wsd, Wst, Wdt, Wtd, eall):
    return pl.pallas_call(
        _rgcn_body,
        grid=(4, EI // BEI),
        in_specs=[
            pl.BlockSpec((G, D), lambda r, c: (0, 0)),
            pl.BlockSpec((G, D), lambda r, c: (0, 0)),
            pl.BlockSpec((D, D), lambda r, c: (0, 0)),
            pl.BlockSpec((D, D), lambda r, c: (0, 0)),
            pl.BlockSpec((D, D), lambda r, c: (0, 0)),
            pl.BlockSpec((D, D), lambda r, c: (0, 0)),
            pl.BlockSpec((1, 2, BEI), lambda r, c: (r, 0, c)),
        ],
        out_specs=pl.BlockSpec((1, G, D), lambda r, c: (r, 0, 0)),
        out_shape=jax.ShapeDtypeStruct((4, G, D), jnp.float32),
    )(gs, gt, Wsd, Wst, Wdt, Wtd, eall)


NC, NS = 2, 16
NW = NC * NS
CH = 32
NBUF = 2
TPW = EP // NW
NCH = TPW // CH
RPT = NP // NS

_sc_mesh = plsc.VectorSubcoreMesh(core_axis_name="c", subcore_axis_name="s")


@functools.partial(
    pl.kernel,
    out_type=[jax.ShapeDtypeStruct((2, NP, D), jnp.float32),
              jax.ShapeDtypeStruct((NW, NP), jnp.float32)],
    mesh=_sc_mesh,
    compiler_params=pltpu.CompilerParams(needs_layout_passes=False),
    scratch_types=[
        pltpu.VMEM((NBUF, CH), jnp.float32),
        pltpu.VMEM((NP,), jnp.float32),
        pltpu.VMEM((D,), jnp.float32),
        pltpu.VMEM((NBUF, CH), jnp.int32),
        pltpu.VMEM((NBUF, CH), jnp.int32),
        pltpu.VMEM((NBUF, CH, D), jnp.float32),
        pltpu.VMEM((NBUF, CH, D), jnp.float32),
        pltpu.VMEM((NBUF, CH, D), jnp.float32),
        pltpu.VMEM((NBUF, CH), jnp.int32),
        pltpu.VMEM_SHARED((NP, D), jnp.float32),
        pltpu.SemaphoreType.DMA,
        pltpu.SemaphoreType.DMA,
        pltpu.SemaphoreType.DMA,
        pltpu.SemaphoreType.DMA,
    ],
)
def _sc_edge(hA, ha, eB, srcp, dstp, attB, acc_out, den_out,
             hav2, den_tab, attb_v, src2, dst2, hag2, ebv2, wmv2, sdst,
             accum_sh, sem0, sem1, sem2, sem3):
    c = lax.axis_index("c")
    s = lax.axis_index("s")
    wid = s * NC + c
    sems = [sem0, sem1]
    ssems = [sem2, sem3]

    pltpu.sync_copy(attB, attb_v)

    def _zrow(r, carry):
        for f in range(8):
            wmv2[0, r, pl.ds(f * 16, 16)] = jnp.zeros((16,), jnp.float32)
        return carry
    lax.fori_loop(0, CH, _zrow, 0)

    def _zden(i, carry):
        den_tab[pl.ds(i * 16, 16)] = jnp.zeros((16,), jnp.float32)
        return carry
    lax.fori_loop(0, NP // 16, _zden, 0)

    for j in range(RPT // CH):
        pltpu.sync_copy(wmv2.at[0], accum_sh.at[pl.ds(s * RPT + j * CH, CH)])
    plsc.subcore_barrier()

    attb_regs = [attb_v[pl.ds(f * 16, 16)] for f in range(8)]
    iota16 = lax.iota(jnp.int32, 16)

    def _start(ci, b):
        base = wid * TPW + ci * CH
        pltpu.sync_copy(srcp.at[pl.ds(base, CH)], src2.at[b])
        pltpu.sync_copy(dstp.at[pl.ds(base, CH)], dst2.at[b])
        pltpu.async_copy(hA.at[src2.at[b]], hag2.at[b], sems[b])
        pltpu.async_copy(eB.at[pl.ds(base, CH)], ebv2.at[b], sems[b])
        pltpu.async_copy(ha.at[dst2.at[b]], hav2.at[b], sems[b])

    def _wait(ci, b):
        base = wid * TPW + ci * CH
        pltpu.make_async_copy(hA.at[src2.at[b]], hag2.at[b], sems[b]).wait()
        pltpu.make_async_copy(eB.at[pl.ds(base, CH)], ebv2.at[b],
                              sems[b]).wait()
        pltpu.make_async_copy(ha.at[dst2.at[b]], hav2.at[b], sems[b]).wait()

    def _compute(b):
        hag, ebv, dst_v = hag2.at[b], ebv2.at[b], dst2.at[b]
        wmv, sdst_b, hav = wmv2.at[b], sdst.at[b], hav2.at[b]
        for g in range(CH // 16):
            dstv = dst_v[pl.ds(g * 16, 16)]
            hadv = hav[pl.ds(g * 16, 16)]
            exacc = jnp.zeros((16,), jnp.float32)
            for j in range(16):
                e = g * 16 + j
                mr = []
                acc = None
                for f in range(8):
                    v = jnp.maximum(
                        hag[e, pl.ds(f * 16, 16)] + ebv[e, pl.ds(f * 16, 16)],
                        0.0)
                    mr.append(v)
                    t = v * attb_regs[f]
                    acc = t if acc is None else acc + t
                sdot = jnp.sum(acc)
                l = hadv[j] + sdot
                l = jnp.where(l >= 0, l, 0.2 * l)
                exv = jnp.exp(jnp.full((16,), l, jnp.float32))
                exacc = jnp.where(iota16 == j, exv, exacc)
                for f in range(8):
                    wmv[e, pl.ds(f * 16, 16)] = mr[f] * exv
            plsc.addupdate_scatter(den_tab, [dstv], exacc)
            sdst_b[pl.ds(g * 16, 16)] = dstv
        pltpu.async_copy(wmv, accum_sh.at[sdst_b], ssems[b], add=True)

    def _scatter_wait(b):
        pltpu.make_async_copy(wmv2.at[b], accum_sh.at[sdst.at[b]],
                              ssems[b]).wait()

    for b in range(NBUF):
        _start(b, b)

    def _body(i, carry):
        for b in range(NBUF):
            cc = i * NBUF + b
            _wait(cc, b)

            @pl.when(cc >= NBUF)
            def _():
                _scatter_wait(b)
            _compute(b)

            @pl.when(cc + NBUF < NCH)
            def _():
                _start(cc + NBUF, b)
        return carry
    lax.fori_loop(0, NCH // NBUF, _body, 0)
    for b in range(NBUF):
        _scatter_wait(b)

    pltpu.sync_copy(den_tab, den_out.at[wid])
    plsc.subcore_barrier()
    for j in range(RPT // CH):
        r0 = s * RPT + j * CH
        pltpu.sync_copy(accum_sh.at[pl.ds(r0, CH)],
                        acc_out.at[c, pl.ds(r0, CH)])


def _attentive_fp(x, edge_index, edge_attr, batch, Wn, We, Wm, att, Wu):
    WmA, WmB = Wm[:D], Wm[D:]
    attA, attB = att[:D], att[D:, 0]
    h, hA, ha = _node_proj(x, Wn, WmA, attA)
    eB = _edge_proj(edge_attr, We, WmB)
    src = edge_index[0].astype(jnp.int32)
    dst = edge_index[1].astype(jnp.int32)
    srcp = jnp.concatenate([src, jnp.zeros((EP - E,), jnp.int32)])
    dstp = jnp.concatenate([dst, jnp.full((EP - E,), N, jnp.int32)])
    hap = jnp.concatenate([ha[:, 0], jnp.zeros((NP - N,), jnp.float32)])
    acc2, den32 = _sc_edge(hA, hap, eB, srcp, dstp, attB)
    batch3 = batch.astype(jnp.int32).reshape(N // BN, 1, BN)
    return _final_readout(h, acc2, den32.T, Wu, batch3)


def kernel(x_small, edge_index_small, edge_attr_small, batch_small,
           x_target, edge_index_target, edge_attr_target, batch_target,
           pos_edge_dt, pos_edge_td, neg_edge_dt, neg_edge_td,
           Wn_s, We_s, Wm_s, att_s, Wu_s,
           Wn_t, We_t, Wm_t, att_t, Wu_t,
           Wsd, Wst, Wdt, Wtd):
    gs = _attentive_fp(x_small, edge_index_small, edge_attr_small,
                       batch_small, Wn_s, We_s, Wm_s, att_s, Wu_s)
    gt = _attentive_fp(x_target, edge_index_target, edge_attr_target,
                       batch_target, Wn_t, We_t, Wm_t, att_t, Wu_t)
    eall = jnp.stack([pos_edge_td, pos_edge_dt, neg_edge_td, neg_edge_dt]
                     ).astype(jnp.int32)
    return _rgcn(gs, gt, Wsd, Wst, Wdt, Wtd, eall)
